# baseline (speedup 1.0000x reference)
import functools

import jax
import jax.numpy as jnp
from jax import lax
from jax.experimental import pallas as pl
from jax.experimental.pallas import tpu as pltpu
from jax.experimental.pallas import tpu_sc as plsc

NC, NS = 2, 16
NW = NC * NS
CH = 256
NBUF = 4
LOOK = NBUF - 2


@functools.lru_cache(maxsize=None)
def _make_gather(per_w, d):
    n_ch = per_w // CH
    mesh = plsc.VectorSubcoreMesh(
        core_axis_name="c", subcore_axis_name="s",
        num_cores=NC, num_subcores=NS,
    )

    @functools.partial(
        pl.kernel,
        out_type=jax.ShapeDtypeStruct((NW, per_w, 2, d // 2), jnp.float32),
        mesh=mesh,
        scratch_types=[
            pltpu.VMEM((per_w,), jnp.int32),
            pltpu.VMEM((NBUF, CH, 2, d // 2), jnp.float32),
        ] + [pltpu.SemaphoreType.DMA] * (2 * NBUF),
        compiler_params=pltpu.CompilerParams(use_tc_tiling_on_sc=False),
    )
    def emb(ids_hbm, table_hbm, out_hbm, idx_v, rows_v, *sems):
        sem_g = sems[:NBUF]
        sem_s = sems[NBUF:]
        wid = lax.axis_index("s") * NC + lax.axis_index("c")
        pltpu.sync_copy(ids_hbm.at[wid], idx_v)

        def gather(j, b):
            return pltpu.make_async_copy(
                table_hbm.at[idx_v.at[pl.ds(j * CH, CH)]], rows_v.at[b], sem_g[b]
            )

        def store(j, b):
            return pltpu.make_async_copy(
                rows_v.at[b], out_hbm.at[wid, pl.ds(j * CH, CH)], sem_s[b]
            )

        for j in range(LOOK):
            gather(j, j % NBUF).start()

        @pl.loop(0, n_ch, step=NBUF)
        def _(g):
            for b in range(NBUF):
                j = g + b
                k = j + LOOK
                kb = (b + LOOK) % NBUF

                @pl.when(k < n_ch)
                def _():
                    @pl.when(k >= NBUF)
                    def _():
                        store(k - NBUF, kb).wait()

                    gather(k, kb).start()

                gather(j, b).wait()
                store(j, b).start()

        for b in range(NBUF):
            j = n_ch - NBUF + b
            store(j, j % NBUF).wait()

    return emb


def kernel(seq_ids, seq_len, table):
    batch, hist = seq_ids.shape
    vocab, d = table.shape
    n_total = batch * hist
    per_w = n_total // NW
    ids = seq_ids.astype(jnp.int32).reshape(NW, per_w)
    table3 = table.reshape(vocab, 2, d // 2)
    out = _make_gather(per_w, d)(ids, table3)
    return out.reshape(batch, hist, d)


# speedup vs baseline: 1.4438x; 1.4438x over previous
import functools

import jax
import jax.numpy as jnp
from jax import lax
from jax.experimental import pallas as pl
from jax.experimental.pallas import tpu as pltpu
from jax.experimental.pallas import tpu_sc as plsc

NC, NS = 2, 16
NW = NC * NS
CHB = 128
L = 16


@functools.lru_cache(maxsize=None)
def _make_emb(t_len, batch, d):
    mesh = plsc.VectorSubcoreMesh(
        core_axis_name="c", subcore_axis_name="s",
        num_cores=NC, num_subcores=NS,
    )
    n_g = CHB // L

    @functools.partial(
        pl.kernel,
        out_type=jax.ShapeDtypeStruct((t_len, d, batch), jnp.float32),
        mesh=mesh,
        scratch_types=[
            pltpu.VMEM((t_len, CHB), jnp.int32),
            pltpu.VMEM((CHB,), jnp.int32),
            pltpu.VMEM((CHB,), jnp.int32),
            pltpu.VMEM((CHB, 2 * d), jnp.float32),
            pltpu.VMEM((CHB, 2 * d), jnp.float32),
            pltpu.VMEM((d, CHB), jnp.float32),
            pltpu.VMEM((d, CHB), jnp.float32),
        ] + [pltpu.SemaphoreType.DMA] * 4,
        compiler_params=pltpu.CompilerParams(
            use_tc_tiling_on_sc=False, needs_layout_passes=False
        ),
    )
    def emb(ids_hbm, tab_hbm, out_hbm, ids_v, pi0, pi1, pr0, pr1, ob0, ob1,
            sg0, sg1, ss0, ss1):
        pidx = (pi0, pi1)
        pairs = (pr0, pr1)
        oblk = (ob0, ob1)
        sem_g = (sg0, sg1)
        sem_s = (ss0, ss1)
        w = lax.axis_index("s") * NC + lax.axis_index("c")
        pltpu.sync_copy(ids_hbm.at[:, pl.ds(w * CHB, CHB)], ids_v)

        iotas = [lax.iota(jnp.int32, L) + g * L for g in range(n_g)]

        def prep(t, s):
            for g in range(n_g):
                v = ids_v[t, pl.ds(g * L, L)]
                pidx[s][pl.ds(g * L, L)] = v >> 1

        def gather(s):
            return pltpu.make_async_copy(
                tab_hbm.at[pidx[s]], pairs[s], sem_g[s]
            )

        def store(t, o):
            return pltpu.make_async_copy(
                oblk[o], out_hbm.at[t, pl.ds(0, d), pl.ds(w * CHB, CHB)],
                sem_s[o],
            )

        def perm(t, s, o):
            half = [(ids_v[t, pl.ds(g * L, L)] & 1) * d for g in range(n_g)]

            @pl.loop(0, d)
            def _(dd):
                for g in range(n_g):
                    oblk[o][dd, pl.ds(g * L, L)] = plsc.load_gather(
                        pairs[s], [iotas[g], half[g] + dd]
                    )

        prep(0, 0)
        gather(0).start()
        prep(1, 1)
        gather(1).start()

        @pl.loop(0, t_len, step=2)
        def _(t0):
            for u in range(2):
                t = t0 + u
                gather(u).wait()
                perm(t, u, u)

                @pl.when(t >= 2)
                def _():
                    store(t - 2, u).wait()

                store(t, u).start()

                @pl.when(t + 2 < t_len)
                def _():
                    prep(t + 2, u)
                    gather(u).start()

        store(t_len - 2, 0).wait()
        store(t_len - 1, 1).wait()

    return emb


def kernel(seq_ids, seq_len, table):
    batch, hist = seq_ids.shape
    vocab, d = table.shape
    ids_t = seq_ids.astype(jnp.int32).T
    table_pairs = table.reshape(vocab // 2, 2 * d)
    out_t = _make_emb(hist, batch, d)(ids_t, table_pairs)
    return out_t.transpose(2, 0, 1)


# speedup vs baseline: 2.9434x; 2.0386x over previous
import functools

import jax
import jax.numpy as jnp
from jax import lax
from jax.experimental import pallas as pl
from jax.experimental.pallas import tpu as pltpu
from jax.experimental.pallas import tpu_sc as plsc

NC, NS = 2, 16
NW = NC * NS
CH = 128
NBUF = 4
LOOK = NBUF - 2
DP = 128


@functools.lru_cache(maxsize=None)
def _make_gather(per_w):
    n_ch = per_w // CH
    mesh = plsc.VectorSubcoreMesh(
        core_axis_name="c", subcore_axis_name="s",
        num_cores=NC, num_subcores=NS,
    )

    @functools.partial(
        pl.kernel,
        out_type=jax.ShapeDtypeStruct((NW, per_w, DP), jnp.float32),
        mesh=mesh,
        scratch_types=[
            pltpu.VMEM((per_w,), jnp.int32),
            pltpu.VMEM((NBUF, CH, DP), jnp.float32),
        ] + [pltpu.SemaphoreType.DMA] * (2 * NBUF),
        compiler_params=pltpu.CompilerParams(use_tc_tiling_on_sc=False),
    )
    def emb(ids_hbm, table_hbm, out_hbm, idx_v, rows_v, *sems):
        sem_g = sems[:NBUF]
        sem_s = sems[NBUF:]
        wid = lax.axis_index("s") * NC + lax.axis_index("c")
        pltpu.sync_copy(ids_hbm.at[wid], idx_v)

        def gather(j, b):
            return pltpu.make_async_copy(
                table_hbm.at[idx_v.at[pl.ds(j * CH, CH)]], rows_v.at[b], sem_g[b]
            )

        def store(j, b):
            return pltpu.make_async_copy(
                rows_v.at[b], out_hbm.at[wid, pl.ds(j * CH, CH)], sem_s[b]
            )

        for j in range(LOOK):
            gather(j, j % NBUF).start()

        @pl.loop(0, n_ch, step=NBUF)
        def _(g):
            for b in range(NBUF):
                j = g + b
                k = j + LOOK
                kb = (b + LOOK) % NBUF

                @pl.when(k < n_ch)
                def _():
                    @pl.when(k >= NBUF)
                    def _():
                        store(k - NBUF, kb).wait()

                    gather(k, kb).start()

                gather(j, b).wait()
                store(j, b).start()

        for b in range(NBUF):
            j = n_ch - NBUF + b
            store(j, j % NBUF).wait()

    return emb


def kernel(seq_ids, seq_len, table):
    batch, hist = seq_ids.shape
    vocab, d = table.shape
    n_total = batch * hist
    per_w = n_total // NW
    ids = seq_ids.astype(jnp.int32).reshape(NW, per_w)
    table_pad = jnp.pad(table, ((0, 0), (0, DP - d)))
    out = _make_gather(per_w)(ids, table_pad)
    return out.reshape(n_total, DP)[:, :d].reshape(batch, hist, d)


# speedup vs baseline: 3.4409x; 1.1690x over previous
import functools

import jax
import jax.numpy as jnp
from jax import lax
from jax.experimental import pallas as pl
from jax.experimental.pallas import tpu as pltpu
from jax.experimental.pallas import tpu_sc as plsc

NC, NS = 2, 16
NW = NC * NS
CH = 256
NBUF = 4
LOOK = NBUF - 2
DP = 128


@functools.lru_cache(maxsize=None)
def _make_gather(per_w):
    n_ch = per_w // CH
    mesh = plsc.VectorSubcoreMesh(
        core_axis_name="c", subcore_axis_name="s",
        num_cores=NC, num_subcores=NS,
    )

    @functools.partial(
        pl.kernel,
        out_type=jax.ShapeDtypeStruct((NW, per_w, DP), jnp.float32),
        mesh=mesh,
        scratch_types=[
            pltpu.VMEM((per_w,), jnp.int32),
            pltpu.VMEM((NBUF, CH, DP // 2), jnp.float32),
        ] + [pltpu.SemaphoreType.DMA] * (2 * NBUF),
        compiler_params=pltpu.CompilerParams(use_tc_tiling_on_sc=False),
    )
    def emb(ids_hbm, table_hbm, out_hbm, idx_v, rows_v, *sems):
        sem_g = sems[:NBUF]
        sem_s = sems[NBUF:]
        wid = lax.axis_index("s") * NC + lax.axis_index("c")
        pltpu.sync_copy(ids_hbm.at[wid], idx_v)

        def gather(j, b):
            return pltpu.make_async_copy(
                table_hbm.at[idx_v.at[pl.ds(j * CH, CH)]], rows_v.at[b], sem_g[b]
            )

        def store(j, b):
            return pltpu.make_async_copy(
                rows_v.at[b],
                out_hbm.at[wid, pl.ds(j * CH, CH), pl.ds(0, DP // 2)],
                sem_s[b],
            )

        for j in range(LOOK):
            gather(j, j % NBUF).start()

        @pl.loop(0, n_ch, step=NBUF)
        def _(g):
            for b in range(NBUF):
                j = g + b
                k = j + LOOK
                kb = (b + LOOK) % NBUF

                @pl.when(k < n_ch)
                def _():
                    @pl.when(k >= NBUF)
                    def _():
                        store(k - NBUF, kb).wait()

                    gather(k, kb).start()

                gather(j, b).wait()
                store(j, b).start()

        for b in range(NBUF):
            j = n_ch - NBUF + b
            store(j, j % NBUF).wait()

    return emb


def kernel(seq_ids, seq_len, table):
    batch, hist = seq_ids.shape
    vocab, d = table.shape
    n_total = batch * hist
    per_w = n_total // NW
    ids = seq_ids.astype(jnp.int32).reshape(NW, per_w) * 2
    table_pad = jnp.pad(table, ((0, 0), (0, DP - d))).reshape(2 * vocab, d)
    out = _make_gather(per_w)(ids, table_pad)
    return out.reshape(n_total, DP)[:, :d].reshape(batch, hist, d)
